# baseline (speedup 1.0000x reference)
import functools

import jax
import jax.numpy as jnp
from jax import lax
from jax.experimental import pallas as pl
from jax.experimental.pallas import tpu as pltpu
from jax.experimental.pallas import tpu_sc as plsc

D = 128
CHUNK = 128
NCH = 25
NBUF = 5

_info = plsc.get_sparse_core_info()
NC = _info.num_cores
NS = _info.num_subcores
NW = NC * NS
B = 100000
SPAN = NCH * CHUNK
LAST = B - CHUNK


def _make_gather():
    mesh = plsc.VectorSubcoreMesh(core_axis_name="c", subcore_axis_name="s")

    @functools.partial(
        pl.kernel,
        mesh=mesh,
        out_type=jax.ShapeDtypeStruct((B, D), jnp.float32),
        scratch_types=[
            pltpu.VMEM((NCH, CHUNK), jnp.int32),
            pltpu.VMEM((NBUF, CHUNK, D), jnp.float32),
            pltpu.SemaphoreType.DMA,
            pltpu.SemaphoreType.DMA((NBUF,)),
            pltpu.SemaphoreType.DMA((NBUF,)),
        ],
    )
    def gather(idx_hbm, table_hbm, out_hbm, idx_v, rows_v, lsem, gsem, ssem):
        wid = lax.axis_index("s") * NC + lax.axis_index("c")
        jbase = wid * NCH

        def chunk_off(i):
            return jnp.minimum((jbase + i) * CHUNK, LAST)

        for r in range(NCH):
            pltpu.async_copy(
                idx_hbm.at[pl.ds(chunk_off(r), CHUNK)], idx_v.at[r], lsem)
        for r in range(NCH):
            pltpu.make_async_copy(
                idx_hbm.at[pl.ds(0, CHUNK)], idx_v.at[r], lsem).wait()

        def start_gather(b, i):
            pltpu.async_copy(table_hbm.at[idx_v.at[i]], rows_v.at[b], gsem.at[b])

        def wait_gather(b):
            pltpu.make_async_copy(
                table_hbm.at[idx_v.at[0]], rows_v.at[b], gsem.at[b]).wait()

        def start_store(b, i):
            pltpu.async_copy(
                rows_v.at[b], out_hbm.at[pl.ds(chunk_off(i), CHUNK)], ssem.at[b])

        def wait_store(b):
            pltpu.make_async_copy(
                rows_v.at[b], out_hbm.at[pl.ds(0, CHUNK)], ssem.at[b]).wait()

        for b in range(NBUF):
            start_gather(b, b)

        def body(k, _):
            for b in range(NBUF):
                wait_gather(b)
                start_store(b, k * NBUF + b)

            @pl.when(k < NCH // NBUF - 1)
            def _():
                for b in range(NBUF):
                    wait_store(b)
                    start_gather(b, (k + 1) * NBUF + b)

            return 0

        lax.fori_loop(0, NCH // NBUF, body, 0)

        for b in range(NBUF):
            wait_store(b)

    return gather


_gather = _make_gather()


def kernel(x, embedding_weight):
    return _gather(x, embedding_weight)


# speedup vs baseline: 1.0869x; 1.0869x over previous
import functools

import jax
import jax.numpy as jnp
from jax import lax
from jax.experimental import pallas as pl
from jax.experimental.pallas import tpu as pltpu
from jax.experimental.pallas import tpu_sc as plsc

D = 128
CHUNK = 128
NBUF = 7

_info = plsc.get_sparse_core_info()
NC = _info.num_cores
NS = _info.num_subcores
NW = NC * NS
B = 100000
NCHT = (B + CHUNK - 1) // CHUNK
NCH_MAX = (NCHT + NW - 1) // NW
LAST = B - CHUNK


def _make_gather():
    mesh = plsc.VectorSubcoreMesh(core_axis_name="c", subcore_axis_name="s")

    @functools.partial(
        pl.kernel,
        mesh=mesh,
        out_type=jax.ShapeDtypeStruct((B, D), jnp.float32),
        scratch_types=[
            pltpu.VMEM((NCH_MAX, CHUNK), jnp.int32),
            pltpu.VMEM((NBUF, CHUNK, D), jnp.float32),
            pltpu.SemaphoreType.DMA,
            pltpu.SemaphoreType.DMA((NBUF,)),
            pltpu.SemaphoreType.DMA((NBUF,)),
        ],
    )
    def gather(idx_hbm, table_hbm, out_hbm, idx_v, rows_v, lsem, gsem, ssem):
        wid = lax.axis_index("s") * NC + lax.axis_index("c")
        jlo = NCHT * wid // NW
        nch = NCHT * (wid + 1) // NW - jlo

        def chunk_off(i):
            return jnp.minimum((jlo + i) * CHUNK, LAST)

        def stage(r, _):
            pltpu.async_copy(
                idx_hbm.at[pl.ds(chunk_off(r), CHUNK)], idx_v.at[r], lsem)
            return 0

        def drain(r, _):
            pltpu.make_async_copy(
                idx_hbm.at[pl.ds(0, CHUNK)], idx_v.at[r], lsem).wait()
            return 0

        def start_gather(s, i):
            pltpu.async_copy(table_hbm.at[idx_v.at[i]], rows_v.at[s], gsem.at[s])

        def wait_gather(s):
            pltpu.make_async_copy(
                table_hbm.at[idx_v.at[0]], rows_v.at[s], gsem.at[s]).wait()

        def start_store(s, i):
            pltpu.async_copy(
                rows_v.at[s], out_hbm.at[pl.ds(chunk_off(i), CHUNK)], ssem.at[s])

        def wait_store(s):
            pltpu.make_async_copy(
                rows_v.at[s], out_hbm.at[pl.ds(0, CHUNK)], ssem.at[s]).wait()

        lax.fori_loop(0, NBUF, stage, 0)
        lax.fori_loop(0, NBUF, drain, 0)
        for b in range(NBUF):
            start_gather(b, b)
        lax.fori_loop(NBUF, nch, stage, 0)
        lax.fori_loop(NBUF, nch, drain, 0)

        def body(i, _):
            s = lax.rem(i, NBUF)
            wait_gather(s)
            start_store(s, i)

            @pl.when(i + NBUF < nch)
            def _():
                wait_store(s)
                start_gather(s, i + NBUF)

            return 0

        lax.fori_loop(0, nch, body, 0)

        for b in range(NBUF):
            wait_store(b)

    return gather


_gather = _make_gather()


def kernel(x, embedding_weight):
    return _gather(x, embedding_weight)


# speedup vs baseline: 1.0960x; 1.0084x over previous
import functools

import jax
import jax.numpy as jnp
from jax import lax
from jax.experimental import pallas as pl
from jax.experimental.pallas import tpu as pltpu
from jax.experimental.pallas import tpu_sc as plsc

D = 128
CHUNK = 128
NBUF = 7

_info = plsc.get_sparse_core_info()
NC = _info.num_cores
NS = _info.num_subcores
NW = NC * NS
B = 100000
NCHT = (B + CHUNK - 1) // CHUNK
NCH_MAX = (NCHT + NW - 1) // NW
LAST = B - CHUNK


def _make_gather():
    mesh = plsc.VectorSubcoreMesh(core_axis_name="c", subcore_axis_name="s")

    @functools.partial(
        pl.kernel,
        mesh=mesh,
        out_type=jax.ShapeDtypeStruct((B, D), jnp.float32),
        scratch_types=[
            pltpu.VMEM((NCH_MAX, CHUNK), jnp.int32),
            pltpu.VMEM((NBUF, CHUNK, D), jnp.float32),
            pltpu.SemaphoreType.DMA,
            pltpu.SemaphoreType.DMA((NBUF,)),
            pltpu.SemaphoreType.DMA((NBUF,)),
        ],
    )
    def gather(idx_hbm, table_hbm, out_hbm, idx_v, rows_v, lsem, gsem, ssem):
        wid = lax.axis_index("s") * NC + lax.axis_index("c")
        jlo = NCHT * wid // NW
        nch = NCHT * (wid + 1) // NW - jlo

        def chunk_off(i):
            return jnp.minimum((jlo + i) * CHUNK, LAST)

        def stage(r, _):
            pltpu.async_copy(
                idx_hbm.at[pl.ds(chunk_off(r), CHUNK)], idx_v.at[r], lsem)
            return 0

        def drain(r, _):
            pltpu.make_async_copy(
                idx_hbm.at[pl.ds(0, CHUNK)], idx_v.at[r], lsem).wait()
            return 0

        def start_gather(s, i):
            pltpu.async_copy(table_hbm.at[idx_v.at[i]], rows_v.at[s], gsem.at[s])

        def wait_gather(s):
            pltpu.make_async_copy(
                table_hbm.at[idx_v.at[0]], rows_v.at[s], gsem.at[s]).wait()

        def start_store(s, i):
            pltpu.async_copy(
                rows_v.at[s], out_hbm.at[pl.ds(chunk_off(i), CHUNK)], ssem.at[s])

        def wait_store(s):
            pltpu.make_async_copy(
                rows_v.at[s], out_hbm.at[pl.ds(0, CHUNK)], ssem.at[s]).wait()

        lax.fori_loop(0, NBUF, stage, 0)
        lax.fori_loop(0, NBUF, drain, 0)
        for b in range(NBUF):
            start_gather(b, b)
        lax.fori_loop(NBUF, nch, stage, 0)
        lax.fori_loop(NBUF, nch, drain, 0)

        def body(i, _):
            s = lax.rem(i, NBUF)
            wait_gather(s)
            start_store(s, i)

            @pl.when(jnp.logical_and(i >= 1, i + NBUF - 1 < nch))
            def _():
                sp = lax.rem(i - 1, NBUF)
                wait_store(sp)
                start_gather(sp, i + NBUF - 1)

            return 0

        lax.fori_loop(0, nch, body, 0)

        for b in range(NBUF):
            wait_store(b)

    return gather


_gather = _make_gather()


def kernel(x, embedding_weight):
    return _gather(x, embedding_weight)


# speedup vs baseline: 1.0960x; 1.0000x over previous
import functools

import jax
import jax.numpy as jnp
from jax import lax
from jax.experimental import pallas as pl
from jax.experimental.pallas import tpu as pltpu
from jax.experimental.pallas import tpu_sc as plsc

D = 128
CHUNK = 128
NBUF = 7

_info = plsc.get_sparse_core_info()
NC = _info.num_cores
NS = _info.num_subcores
NW = NC * NS
B = 100000
NCHT = (B + CHUNK - 1) // CHUNK
NCH_MAX = (NCHT + NW - 1) // NW
LAST = B - CHUNK


def _make_gather():
    mesh = plsc.VectorSubcoreMesh(core_axis_name="c", subcore_axis_name="s")

    @functools.partial(
        pl.kernel,
        mesh=mesh,
        out_type=jax.ShapeDtypeStruct((B, D), jnp.float32),
        scratch_types=[
            pltpu.VMEM((NCH_MAX, CHUNK), jnp.int32),
            pltpu.VMEM((NBUF, CHUNK, D), jnp.float32),
            pltpu.SemaphoreType.DMA,
            pltpu.SemaphoreType.DMA((NBUF,)),
            pltpu.SemaphoreType.DMA((NBUF,)),
        ],
    )
    def gather(idx_hbm, table_hbm, out_hbm, idx_v, rows_v, lsem, gsem, ssem):
        wid = lax.axis_index("s") * NC + lax.axis_index("c")
        jlo = NCHT * wid // NW
        nch = NCHT * (wid + 1) // NW - jlo

        def chunk_off(i):
            return jnp.minimum((jlo + i) * CHUNK, LAST)

        def stage(r, _):
            pltpu.async_copy(
                idx_hbm.at[pl.ds(chunk_off(r), CHUNK)], idx_v.at[r], lsem)
            return 0

        def drain(r, _):
            pltpu.make_async_copy(
                idx_hbm.at[pl.ds(0, CHUNK)], idx_v.at[r], lsem).wait()
            return 0

        def start_gather(s, i):
            pltpu.async_copy(table_hbm.at[idx_v.at[i]], rows_v.at[s], gsem.at[s])

        def wait_gather(s):
            pltpu.make_async_copy(
                table_hbm.at[idx_v.at[0]], rows_v.at[s], gsem.at[s]).wait()

        def start_store(s, i):
            off = chunk_off(i)
            for h in range(2):
                pltpu.async_copy(
                    rows_v.at[s, pl.ds(h * (CHUNK // 2), CHUNK // 2)],
                    out_hbm.at[pl.ds(off + h * (CHUNK // 2), CHUNK // 2)],
                    ssem.at[s])

        def wait_store(s):
            pltpu.make_async_copy(
                rows_v.at[s], out_hbm.at[pl.ds(0, CHUNK)], ssem.at[s]).wait()

        lax.fori_loop(0, NBUF, stage, 0)
        lax.fori_loop(0, NBUF, drain, 0)
        for b in range(NBUF):
            start_gather(b, b)
        lax.fori_loop(NBUF, nch, stage, 0)
        lax.fori_loop(NBUF, nch, drain, 0)

        def body(i, _):
            s = lax.rem(i, NBUF)
            wait_gather(s)
            start_store(s, i)

            @pl.when(jnp.logical_and(i >= 1, i + NBUF - 1 < nch))
            def _():
                sp = lax.rem(i - 1, NBUF)
                wait_store(sp)
                start_gather(sp, i + NBUF - 1)

            return 0

        lax.fori_loop(0, nch, body, 0)

        for b in range(NBUF):
            wait_store(b)

    return gather


_gather = _make_gather()


def kernel(x, embedding_weight):
    return _gather(x, embedding_weight)
